# Initial kernel scaffold; baseline (speedup 1.0000x reference)
import jax
import jax.numpy as jnp
from jax.experimental import pallas as pl


def kernel(x, edge_index, W1_l, b1_l, W1_r, W2_l, b2_l, W2_r):
    raise NotImplementedError("write your pallas kernel here")


# speedup vs baseline: 2.4389x; 2.4389x over previous
import jax
import jax.numpy as jnp
from jax import lax
from jax.experimental import pallas as pl
from jax.experimental.pallas import tpu as pltpu
from jax.experimental.pallas import tpu_sc as plsc

N = 10000
D = 128
E = 320000

NC = 2
NS = 16
NW = NC * NS

R = 10240
EPT = 10240
EPAD = NW * EPT
C = 128
ITERS = EPT // C
ZR = R // NS

BR = 1280
GRID = R // BR


def _make_sc_scatter():
  mesh = plsc.VectorSubcoreMesh(core_axis_name="c", subcore_axis_name="s")

  def body(y_h, src_h, dst_h, z_h, out_h, src_v, dst_v, rows_v, acc, sem):
    c = lax.axis_index("c")
    s = lax.axis_index("s")
    ebase = (c * NS + s) * EPT
    zb = s * ZR
    nzc = ZR // C

    pltpu.sync_copy(z_h, rows_v)
    for k in range(nzc):
      pltpu.sync_copy(rows_v, acc.at[pl.ds(zb + k * C, C)])
    plsc.subcore_barrier()

    def step(i, carry):
      base = ebase + i * C
      pltpu.sync_copy(src_h.at[pl.ds(base, C)], src_v)
      pltpu.sync_copy(dst_h.at[pl.ds(base, C)], dst_v)
      pltpu.async_copy(y_h.at[src_v], rows_v, sem).wait()
      pltpu.sync_copy(rows_v, acc.at[dst_v], add=True)
      return carry

    lax.fori_loop(0, ITERS, step, 0)
    plsc.subcore_barrier()

    for k in range(nzc):
      pltpu.sync_copy(acc.at[pl.ds(zb + k * C, C)], rows_v)
      pltpu.sync_copy(rows_v, out_h.at[pl.ds(c * R + zb + k * C, C)])

  return pl.kernel(
      body,
      out_type=jax.ShapeDtypeStruct((NC * R, D), jnp.float32),
      mesh=mesh,
      scratch_types=[
          pltpu.VMEM((C,), jnp.int32),
          pltpu.VMEM((C,), jnp.int32),
          pltpu.VMEM((C, D), jnp.float32),
          pltpu.VMEM_SHARED((R, D), jnp.float32),
          pltpu.SemaphoreType.DMA,
      ])


def _make_sc_deg():
  mesh = plsc.VectorSubcoreMesh(core_axis_name="c", subcore_axis_name="s")

  def body(dst_h, z_h, one_h, deg_h, dst_v, stg_v, ones_v, acc):
    c = lax.axis_index("c")
    s = lax.axis_index("s")
    ebase = (c * NS + s) * EPT
    zb = s * ZR
    nzc = ZR // C

    pltpu.sync_copy(z_h, stg_v)
    pltpu.sync_copy(one_h, ones_v)
    for k in range(nzc):
      pltpu.sync_copy(stg_v, acc.at[pl.ds(zb + k * C, C)])
    plsc.subcore_barrier()

    def step(i, carry):
      pltpu.sync_copy(dst_h.at[pl.ds(ebase + i * C, C)], dst_v)
      pltpu.sync_copy(ones_v, acc.at[dst_v], add=True)
      return carry

    lax.fori_loop(0, ITERS, step, 0)
    plsc.subcore_barrier()

    for k in range(nzc):
      pltpu.sync_copy(acc.at[pl.ds(zb + k * C, C)], stg_v)
      pltpu.sync_copy(stg_v, deg_h.at[pl.ds(c * R + zb + k * C, C)])

  return pl.kernel(
      body,
      out_type=jax.ShapeDtypeStruct((NC * R, D), jnp.float32),
      mesh=mesh,
      scratch_types=[
          pltpu.VMEM((C,), jnp.int32),
          pltpu.VMEM((C, D), jnp.float32),
          pltpu.VMEM((C, D), jnp.float32),
          pltpu.VMEM_SHARED((R, D), jnp.float32),
      ])


_sc_scatter = _make_sc_scatter()
_sc_deg = _make_sc_deg()


def _mm2_body(x_ref, wl_ref, wr_ref, yl_ref, yr_ref):
  x = x_ref[...]
  yl_ref[...] = jnp.dot(x, wl_ref[...], preferred_element_type=jnp.float32)
  yr_ref[...] = jnp.dot(x, wr_ref[...], preferred_element_type=jnp.float32)


def _mid_body(s_ref, deg_ref, r_ref, b_ref, wl_ref, wr_ref, yl_ref, yr_ref):
  s = s_ref[0] + s_ref[1]
  h = jnp.maximum(s / deg_ref[...] + r_ref[...] + b_ref[...], 0.0)
  yl_ref[...] = jnp.dot(h, wl_ref[...], preferred_element_type=jnp.float32)
  yr_ref[...] = jnp.dot(h, wr_ref[...], preferred_element_type=jnp.float32)


def _out_body(s_ref, deg_ref, r_ref, b_ref, o_ref):
  s = s_ref[0] + s_ref[1]
  o_ref[...] = jax.nn.sigmoid(s / deg_ref[...] + r_ref[...] + b_ref[...])


_row_spec = pl.BlockSpec((BR, D), lambda i: (i, 0))
_w_spec = pl.BlockSpec((D, D), lambda i: (0, 0))
_b_spec = pl.BlockSpec((1, D), lambda i: (0, 0))
_acc_spec = pl.BlockSpec((NC, BR, D), lambda i: (0, i, 0))
_deg_spec = pl.BlockSpec((BR, 1), lambda i: (i, 0))
_rowD = jax.ShapeDtypeStruct((R, D), jnp.float32)

_tc_mm2 = pl.pallas_call(
    _mm2_body, grid=(GRID,),
    in_specs=[_row_spec, _w_spec, _w_spec],
    out_specs=[_row_spec, _row_spec],
    out_shape=[_rowD, _rowD])

_tc_mid = pl.pallas_call(
    _mid_body, grid=(GRID,),
    in_specs=[_acc_spec, _deg_spec, _row_spec, _b_spec, _w_spec, _w_spec],
    out_specs=[_row_spec, _row_spec],
    out_shape=[_rowD, _rowD])

_tc_out = pl.pallas_call(
    _out_body, grid=(GRID,),
    in_specs=[_acc_spec, _deg_spec, _row_spec, _b_spec],
    out_specs=_row_spec,
    out_shape=_rowD)


@jax.jit
def kernel(x, edge_index, W1_l, b1_l, W1_r, W2_l, b2_l, W2_r):
  xp = jnp.zeros((R, D), x.dtype).at[:N].set(x)
  src = jnp.zeros((EPAD,), jnp.int32).at[:E].set(edge_index[0])
  dst = jnp.full((EPAD,), N, jnp.int32).at[:E].set(edge_index[1])
  z128 = jnp.zeros((C, D), jnp.float32)
  b1 = b1_l.reshape(1, D)
  b2 = b2_l.reshape(1, D)

  ones128 = jnp.ones((C, D), jnp.float32)

  degp = _sc_deg(dst, z128, ones128).reshape(NC, R, D)
  deg = jnp.maximum(degp[0, :, 0] + degp[1, :, 0], 1.0).reshape(R, 1)
  y1, r1 = _tc_mm2(xp, W1_l, W1_r)
  s1 = _sc_scatter(y1, src, dst, z128).reshape(NC, R, D)
  y2, r2 = _tc_mid(s1, deg, r1, b1, W2_l, W2_r)
  s2 = _sc_scatter(y2, src, dst, z128).reshape(NC, R, D)
  out = _tc_out(s2, deg, r2, b2)
  return out[:N]


# speedup vs baseline: 3.4603x; 1.4188x over previous
import jax
import jax.numpy as jnp
from jax import lax
from jax.experimental import pallas as pl
from jax.experimental.pallas import tpu as pltpu
from jax.experimental.pallas import tpu_sc as plsc

N = 10000
D = 128
E = 320000

NC = 2
NS = 16
NW = NC * NS

R = 10240
EPT = 10240
EPAD = NW * EPT
C = 128
ITERS = EPT // C
ZR = R // NS

BR = 1280
GRID = R // BR


def _make_sc_scatter():
  mesh = plsc.VectorSubcoreMesh(core_axis_name="c", subcore_axis_name="s")

  def body(y_h, sd_h, z_h, out_h, sd0, sd1, r0, r1, acc, sem0, sem1):
    c = lax.axis_index("c")
    s = lax.axis_index("s")
    cbase = (c * NS + s) * ITERS
    zb = s * ZR
    nzc = ZR // C

    pltpu.sync_copy(z_h, r0)
    for k in range(nzc):
      pltpu.sync_copy(r0, acc.at[pl.ds(zb + k * C, C)])
    plsc.subcore_barrier()

    pltpu.sync_copy(sd_h.at[cbase], sd0)
    pltpu.async_copy(y_h.at[sd0.at[0]], r0, sem0)

    def step2(j, carry):
      i = cbase + 2 * j
      pltpu.sync_copy(sd_h.at[i + 1], sd1)
      pltpu.async_copy(y_h.at[sd1.at[0]], r1, sem1)
      pltpu.make_async_copy(y_h.at[sd0.at[0]], r0, sem0).wait()
      pltpu.sync_copy(r0, acc.at[sd0.at[1]], add=True)
      nxt = jnp.minimum(i + 2, cbase + ITERS - 1)
      pltpu.sync_copy(sd_h.at[nxt], sd0)
      pltpu.async_copy(y_h.at[sd0.at[0]], r0, sem0)
      pltpu.make_async_copy(y_h.at[sd1.at[0]], r1, sem1).wait()
      pltpu.sync_copy(r1, acc.at[sd1.at[1]], add=True)
      return carry

    lax.fori_loop(0, ITERS // 2, step2, 0)
    pltpu.make_async_copy(y_h.at[sd0.at[0]], r0, sem0).wait()
    plsc.subcore_barrier()

    for k in range(nzc):
      pltpu.sync_copy(acc.at[pl.ds(zb + k * C, C)], r0)
      pltpu.sync_copy(r0, out_h.at[pl.ds(c * R + zb + k * C, C)])

  return pl.kernel(
      body,
      out_type=jax.ShapeDtypeStruct((NC * R, D), jnp.float32),
      mesh=mesh,
      scratch_types=[
          pltpu.VMEM((2, C), jnp.int32),
          pltpu.VMEM((2, C), jnp.int32),
          pltpu.VMEM((C, D), jnp.float32),
          pltpu.VMEM((C, D), jnp.float32),
          pltpu.VMEM_SHARED((R, D), jnp.float32),
          pltpu.SemaphoreType.DMA,
          pltpu.SemaphoreType.DMA,
      ])


def _make_sc_deg():
  mesh = plsc.VectorSubcoreMesh(core_axis_name="c", subcore_axis_name="s")

  def body(sd_h, z_h, one_h, deg_h, sd0, sd1, stg_v, ones_v, acc, sem0, sem1):
    c = lax.axis_index("c")
    s = lax.axis_index("s")
    cbase = (c * NS + s) * ITERS
    zb = s * ZR
    nzc = ZR // C

    pltpu.sync_copy(z_h, stg_v)
    pltpu.sync_copy(one_h, ones_v)
    for k in range(nzc):
      pltpu.sync_copy(stg_v, acc.at[pl.ds(zb + k * C, C)])
    plsc.subcore_barrier()

    pltpu.async_copy(sd_h.at[cbase], sd0, sem0)

    def step2(j, carry):
      i = cbase + 2 * j
      pltpu.async_copy(sd_h.at[i + 1], sd1, sem1)
      pltpu.make_async_copy(sd_h.at[i], sd0, sem0).wait()
      pltpu.sync_copy(ones_v, acc.at[sd0.at[1]], add=True)
      nxt = jnp.minimum(i + 2, cbase + ITERS - 1)
      pltpu.async_copy(sd_h.at[nxt], sd0, sem0)
      pltpu.make_async_copy(sd_h.at[i + 1], sd1, sem1).wait()
      pltpu.sync_copy(ones_v, acc.at[sd1.at[1]], add=True)
      return carry

    lax.fori_loop(0, ITERS // 2, step2, 0)
    pltpu.make_async_copy(sd_h.at[cbase], sd0, sem0).wait()
    plsc.subcore_barrier()

    for k in range(nzc):
      pltpu.sync_copy(acc.at[pl.ds(zb + k * C, C)], stg_v)
      pltpu.sync_copy(stg_v, deg_h.at[pl.ds(c * R + zb + k * C, C)])

  return pl.kernel(
      body,
      out_type=jax.ShapeDtypeStruct((NC * R, D), jnp.float32),
      mesh=mesh,
      scratch_types=[
          pltpu.VMEM((2, C), jnp.int32),
          pltpu.VMEM((2, C), jnp.int32),
          pltpu.VMEM((C, D), jnp.float32),
          pltpu.VMEM((C, D), jnp.float32),
          pltpu.VMEM_SHARED((R, D), jnp.float32),
          pltpu.SemaphoreType.DMA,
          pltpu.SemaphoreType.DMA,
      ])


_sc_scatter = _make_sc_scatter()
_sc_deg = _make_sc_deg()


def _mm2_body(x_ref, wl_ref, wr_ref, yl_ref, yr_ref):
  x = x_ref[...]
  yl_ref[...] = jnp.dot(x, wl_ref[...], preferred_element_type=jnp.float32)
  yr_ref[...] = jnp.dot(x, wr_ref[...], preferred_element_type=jnp.float32)


def _mid_body(s_ref, deg_ref, r_ref, b_ref, wl_ref, wr_ref, yl_ref, yr_ref):
  s = s_ref[0] + s_ref[1]
  h = jnp.maximum(s / deg_ref[...] + r_ref[...] + b_ref[...], 0.0)
  yl_ref[...] = jnp.dot(h, wl_ref[...], preferred_element_type=jnp.float32)
  yr_ref[...] = jnp.dot(h, wr_ref[...], preferred_element_type=jnp.float32)


def _out_body(s_ref, deg_ref, r_ref, b_ref, o_ref):
  s = s_ref[0] + s_ref[1]
  o_ref[...] = jax.nn.sigmoid(s / deg_ref[...] + r_ref[...] + b_ref[...])


_row_spec = pl.BlockSpec((BR, D), lambda i: (i, 0))
_w_spec = pl.BlockSpec((D, D), lambda i: (0, 0))
_b_spec = pl.BlockSpec((1, D), lambda i: (0, 0))
_acc_spec = pl.BlockSpec((NC, BR, D), lambda i: (0, i, 0))
_deg_spec = pl.BlockSpec((BR, 1), lambda i: (i, 0))
_rowD = jax.ShapeDtypeStruct((R, D), jnp.float32)

_tc_mm2 = pl.pallas_call(
    _mm2_body, grid=(GRID,),
    in_specs=[_row_spec, _w_spec, _w_spec],
    out_specs=[_row_spec, _row_spec],
    out_shape=[_rowD, _rowD])

_tc_mid = pl.pallas_call(
    _mid_body, grid=(GRID,),
    in_specs=[_acc_spec, _deg_spec, _row_spec, _b_spec, _w_spec, _w_spec],
    out_specs=[_row_spec, _row_spec],
    out_shape=[_rowD, _rowD])

_tc_out = pl.pallas_call(
    _out_body, grid=(GRID,),
    in_specs=[_acc_spec, _deg_spec, _row_spec, _b_spec],
    out_specs=_row_spec,
    out_shape=_rowD)


@jax.jit
def kernel(x, edge_index, W1_l, b1_l, W1_r, W2_l, b2_l, W2_r):
  xp = jnp.zeros((R, D), x.dtype).at[:N].set(x)
  src = jnp.zeros((EPAD,), jnp.int32).at[:E].set(edge_index[0])
  dst = jnp.full((EPAD,), N, jnp.int32).at[:E].set(edge_index[1])
  sd = jnp.concatenate(
      [src.reshape(-1, 1, C), dst.reshape(-1, 1, C)], axis=1)
  z128 = jnp.zeros((C, D), jnp.float32)
  b1 = b1_l.reshape(1, D)
  b2 = b2_l.reshape(1, D)

  ones128 = jnp.ones((C, D), jnp.float32)

  degp = _sc_deg(sd, z128, ones128).reshape(NC, R, D)
  deg = jnp.maximum(degp[0, :, 0] + degp[1, :, 0], 1.0).reshape(R, 1)
  y1, r1 = _tc_mm2(xp, W1_l, W1_r)
  s1 = _sc_scatter(y1, sd, z128).reshape(NC, R, D)
  y2, r2 = _tc_mid(s1, deg, r1, b1, W2_l, W2_r)
  s2 = _sc_scatter(y2, sd, z128).reshape(NC, R, D)
  out = _tc_out(s2, deg, r2, b2)
  return out[:N]
